# baseline (speedup 1.0000x reference)
import functools

import jax
import jax.numpy as jnp
from jax import lax
from jax.experimental import pallas as pl
from jax.experimental.pallas import tpu as pltpu
from jax.experimental.pallas import tpu_sc as plsc

_NUM_ROWS = 100000
_BATCH = 16384
_D = 2
_TBL = _NUM_ROWS * _D
_N = _BATCH * _D

_info = plsc.get_sparse_core_info()
_NC, _NS = _info.num_cores, _info.num_subcores
_NW = _NC * _NS
_E_PER_W = _N // _NW
_STAGE_PER_TILE = 12496
_TAIL_BASE = _STAGE_PER_TILE * _NS
_TAIL_ELEMS = _TBL - _TAIL_BASE

_mesh = plsc.VectorSubcoreMesh(core_axis_name="c", subcore_axis_name="s")


@functools.partial(
    pl.kernel,
    mesh=_mesh,
    out_type=jax.ShapeDtypeStruct((_N,), jnp.float32),
    scratch_types=[
        pltpu.VMEM((_E_PER_W,), jnp.int32),
        pltpu.VMEM((_E_PER_W,), jnp.float32),
        pltpu.VMEM((_STAGE_PER_TILE,), jnp.float32),
        pltpu.VMEM_SHARED((_TBL,), jnp.float32),
        pltpu.SemaphoreType.DMA,
    ],
)
def _gather_flat(idx_hbm, table_hbm, out_hbm, idx_v, vals_v, stage_v, tbl_sh, sem):
    sid = lax.axis_index("s")
    wid = sid * _NC + lax.axis_index("c")
    base = wid * _E_PER_W
    pltpu.sync_copy(idx_hbm.at[pl.ds(base, _E_PER_W)], idx_v)
    chunk = pl.ds(sid * _STAGE_PER_TILE, _STAGE_PER_TILE)
    pltpu.sync_copy(table_hbm.at[chunk], stage_v)
    pltpu.sync_copy(stage_v, tbl_sh.at[chunk])

    @pl.when(sid == 0)
    def _stage_tail():
        tail = pl.ds(_TAIL_BASE, _TAIL_ELEMS)
        pltpu.sync_copy(table_hbm.at[tail], stage_v.at[pl.ds(0, _TAIL_ELEMS)])
        pltpu.sync_copy(stage_v.at[pl.ds(0, _TAIL_ELEMS)], tbl_sh.at[tail])

    plsc.subcore_barrier()
    pltpu.async_copy(tbl_sh.at[idx_v], vals_v, sem).wait()
    pltpu.sync_copy(vals_v, out_hbm.at[pl.ds(base, _E_PER_W)])


def kernel(x, id_to_gps):
    xi = x.astype(jnp.int32)
    flat_idx = (xi[:, None] * _D + jnp.arange(_D, dtype=jnp.int32)).reshape(-1)
    out = _gather_flat(flat_idx, id_to_gps.reshape(-1))
    return out.reshape(_BATCH, _D)
